# Initial kernel scaffold; baseline (speedup 1.0000x reference)
import jax
import jax.numpy as jnp
from jax.experimental import pallas as pl


def kernel(query, key, lsh_proj, head_idx):
    raise NotImplementedError("write your pallas kernel here")


# speedup vs baseline: 31.1748x; 31.1748x over previous
import jax
import jax.numpy as jnp
from jax import lax
from jax.experimental import pallas as pl
from jax.experimental.pallas import tpu as pltpu
from jax.experimental.pallas import tpu_sc as plsc

LSH_BUCKETS = 64
LSH_BANDWIDTH = 2.0
N_HASH = 4
K_MAX = 32

NUM_CORES = 2
NUM_SUBCORES = 16
LANES = 16
NW = NUM_CORES * NUM_SUBCORES

NB = 256


def _bucket_hash(p):
    x = p / LSH_BANDWIDTH
    t = x.astype(jnp.int32)
    tf = t.astype(jnp.float32)
    f = t - jnp.where((x < 0.0) & (tf != x), 1, 0)
    return ((f % LSH_BUCKETS) + LSH_BUCKETS) % LSH_BUCKETS


def _fold(c):
    return (c ^ (c >> 8) ^ (c >> 16)) & (NB - 1)


def _make_sc_call(B, L):
    n = B * L
    qpw = n // NW
    assert qpw * NW == n and L % LANES == 0 and qpw % LANES == 0
    assert L & (L - 1) == 0
    log2l = L.bit_length() - 1

    def body(qproj_hbm, kproj_hbm, out_hbm, qp_v, kp_v, qc_v, kc_v, hist_v,
             offs_v, end_v, cur_v, kpack_s, buf_v, out_v, sem):
        wid = lax.axis_index("s") * NUM_CORES + lax.axis_index("c")
        gbase = wid * qpw
        b = gbase // L
        kbase = b * L

        copies = []
        for h in range(N_HASH):
            copies.append(pltpu.async_copy(
                qproj_hbm.at[pl.ds(h * n + gbase, qpw)],
                qp_v.at[pl.ds(h * qpw, qpw)], sem))
            copies.append(pltpu.async_copy(
                kproj_hbm.at[pl.ds(h * n + kbase, L)],
                kp_v.at[pl.ds(h * L, L)], sem))
        for c in copies:
            c.wait()

        def pack_codes(src, dst, count):
            def step(i, _):
                hs = [_bucket_hash(src[pl.ds(h * count + i * LANES, LANES)])
                      for h in range(N_HASH)]
                c = ((hs[3] * LSH_BUCKETS + hs[2]) * LSH_BUCKETS
                     + hs[1]) * LSH_BUCKETS + hs[0]
                dst[pl.ds(i * LANES, LANES)] = c
                return 0
            lax.fori_loop(0, count // LANES, step, 0)

        pack_codes(qp_v, qc_v, qpw)
        pack_codes(kp_v, kc_v, L)

        iota = lax.iota(jnp.int32, LANES)
        zeros = jnp.zeros((LANES,), jnp.int32)

        def zero_step(j, _):
            hist_v[pl.ds(j * LANES, LANES)] = zeros
            return 0
        lax.fori_loop(0, NB // LANES, zero_step, 0)

        def hist_step(i, _):
            bv = _fold(kc_v[pl.ds(i * LANES, LANES)])
            rank, lastm = plsc.scan_count(bv)
            plsc.addupdate_scatter(hist_v, [bv], rank, mask=lastm)
            return 0
        lax.fori_loop(0, L // LANES, hist_step, 0)

        def prefix_step(j, carry):
            hv = hist_v[pl.ds(j * LANES, LANES)]
            cs = plsc.cumsum(hv)
            offs_v[pl.ds(j * LANES, LANES)] = carry + cs - hv
            end_v[pl.ds(j * LANES, LANES)] = carry + cs
            cur_v[pl.ds(j * LANES, LANES)] = carry + cs - hv
            return carry + jnp.max(cs)
        lax.fori_loop(0, NB // LANES, prefix_step, 0)

        def place_step(i, _):
            kvv = kc_v[pl.ds(i * LANES, LANES)]
            bv = _fold(kvv)
            rank, lastm = plsc.scan_count(bv)
            base = plsc.load_gather(cur_v, [bv])
            pos = base + rank - 1
            packed = ((kvv >> 8) << log2l) | (iota + i * LANES)
            plsc.store_scatter(kpack_s, [pos], packed)
            plsc.addupdate_scatter(cur_v, [bv], rank, mask=lastm)
            return 0
        lax.fori_loop(0, L // LANES, place_step, 0)

        def per_query(q, _):
            qsplat = plsc.load_gather(qc_v, [jnp.full((LANES,), q, jnp.int32)])
            qbv = _fold(qsplat)
            qtop = (qsplat >> 8) << log2l
            endv = plsc.load_gather(end_v, [qbv])
            start = jnp.max(plsc.load_gather(offs_v, [qbv]))
            end = jnp.max(endv)

            def cond(carry):
                ptr, cnt = carry
                return (ptr < end) & (cnt < K_MAX)

            def loop(carry):
                ptr, cnt = carry
                idx16 = ptr + iota
                v = plsc.load_gather(kpack_s, [jnp.minimum(idx16, L - 1)])
                m = (idx16 < endv) & ((v & (-L)) == qtop)
                plsc.store_compressed(buf_v.at[pl.ds(cnt, LANES)], v, mask=m)
                return ptr + LANES, cnt + jnp.sum(m.astype(jnp.int32))

            _, cnt = lax.while_loop(cond, loop, (start, 0))
            count = jnp.minimum(cnt, K_MAX)
            row0 = jnp.where(iota < count,
                             buf_v[pl.ds(0, LANES)] & (L - 1), -1)
            row1 = jnp.where(iota + LANES < count,
                             buf_v[pl.ds(LANES, LANES)] & (L - 1), -1)
            out_v[pl.ds(q * K_MAX, LANES)] = row0
            out_v[pl.ds(q * K_MAX + LANES, LANES)] = row1
            return 0

        lax.fori_loop(0, qpw, per_query, 0)
        pltpu.sync_copy(out_v, out_hbm.at[pl.ds(gbase * K_MAX, qpw * K_MAX)])

    return pl.kernel(
        body,
        out_type=jax.ShapeDtypeStruct((n * K_MAX,), jnp.int32),
        mesh=plsc.VectorSubcoreMesh(core_axis_name="c", subcore_axis_name="s",
                                    num_cores=NUM_CORES,
                                    num_subcores=NUM_SUBCORES),
        scratch_types=[
            pltpu.VMEM((N_HASH * qpw,), jnp.float32),
            pltpu.VMEM((N_HASH * L,), jnp.float32),
            pltpu.VMEM((qpw,), jnp.int32),
            pltpu.VMEM((L,), jnp.int32),
            pltpu.VMEM((NB,), jnp.int32),
            pltpu.VMEM((NB,), jnp.int32),
            pltpu.VMEM((NB,), jnp.int32),
            pltpu.VMEM((NB,), jnp.int32),
            pltpu.VMEM((L,), jnp.int32),
            pltpu.VMEM((4 * LANES,), jnp.int32),
            pltpu.VMEM((qpw * K_MAX,), jnp.int32),
            pltpu.SemaphoreType.DMA,
        ],
        compiler_params=pltpu.CompilerParams(needs_layout_passes=False,
                                             skip_device_barrier=True),
    )


@jax.jit
def kernel(query, key, lsh_proj, head_idx=0):
    B, L, D = query.shape
    n = B * L
    query_bin = (query > 0).astype(jnp.float32)
    key_bin = (key > 0).astype(jnp.float32)
    qproj = query_bin.reshape(n, -1) @ lsh_proj
    kproj = key_bin.reshape(n, -1) @ lsh_proj
    qproj_t = qproj.T.reshape(-1)
    kproj_t = kproj.T.reshape(-1)
    out = _make_sc_call(B, L)(qproj_t, kproj_t)
    return out.reshape(B, L, K_MAX)


# speedup vs baseline: 33.0186x; 1.0591x over previous
import jax
import jax.numpy as jnp
from jax import lax
from jax.experimental import pallas as pl
from jax.experimental.pallas import tpu as pltpu
from jax.experimental.pallas import tpu_sc as plsc

LSH_BUCKETS = 64
LSH_BANDWIDTH = 2.0
N_HASH = 4
K_MAX = 32

NUM_CORES = 2
NUM_SUBCORES = 16
LANES = 16
NW = NUM_CORES * NUM_SUBCORES

NB = 256


def _bucket_hash(p):
    x = p / LSH_BANDWIDTH
    t = x.astype(jnp.int32)
    tf = t.astype(jnp.float32)
    f = t - jnp.where((x < 0.0) & (tf != x), 1, 0)
    return ((f % LSH_BUCKETS) + LSH_BUCKETS) % LSH_BUCKETS


def _fold(c):
    return (c ^ (c >> 8) ^ (c >> 16)) & (NB - 1)


def _make_sc_call(B, L):
    n = B * L
    qpw = n // NW
    assert qpw * NW == n and L % LANES == 0 and qpw % LANES == 0
    assert L & (L - 1) == 0
    log2l = L.bit_length() - 1

    def body(qproj_hbm, kproj_hbm, out_hbm, qp_v, kp_v, qc_v, kc_v, hist_v,
             comb_v, cur_v, kpack_s, cnt_s, out_v, sem):
        wid = lax.axis_index("s") * NUM_CORES + lax.axis_index("c")
        gbase = wid * qpw
        b = gbase // L
        kbase = b * L
        lbase = gbase - kbase

        copies = [
            pltpu.async_copy(qproj_hbm.at[pl.ds(gbase * N_HASH,
                                                qpw * N_HASH)], qp_v, sem),
            pltpu.async_copy(kproj_hbm.at[pl.ds(kbase * N_HASH,
                                                L * N_HASH)], kp_v, sem),
        ]
        for c in copies:
            c.wait()

        def pack_codes(src, dst, count):
            base_idx = lax.iota(jnp.int32, LANES) * N_HASH

            def step(i, _):
                hs = [_bucket_hash(plsc.load_gather(
                          src, [base_idx + (i * LANES * N_HASH + h)]))
                      for h in range(N_HASH)]
                c = ((hs[3] * LSH_BUCKETS + hs[2]) * LSH_BUCKETS
                     + hs[1]) * LSH_BUCKETS + hs[0]
                dst[pl.ds(i * LANES, LANES)] = c
                return 0
            lax.fori_loop(0, count // LANES, step, 0)

        pack_codes(qp_v, qc_v, qpw)
        pack_codes(kp_v, kc_v, L)

        iota = lax.iota(jnp.int32, LANES)
        zeros = jnp.zeros((LANES,), jnp.int32)

        def zero_step(j, _):
            hist_v[pl.ds(j * LANES, LANES)] = zeros
            return 0
        lax.fori_loop(0, NB // LANES, zero_step, 0)

        def hist_step(i, _):
            bv = _fold(kc_v[pl.ds(i * LANES, LANES)])
            rank, lastm = plsc.scan_count(bv)
            plsc.addupdate_scatter(hist_v, [bv], rank, mask=lastm)
            return 0
        lax.fori_loop(0, L // LANES, hist_step, 0)

        def prefix_step(j, carry):
            hv = hist_v[pl.ds(j * LANES, LANES)]
            cs = plsc.cumsum(hv)
            excl = carry + cs - hv
            comb_v[pl.ds(j * LANES, LANES)] = (excl << 12) | hv
            cur_v[pl.ds(j * LANES, LANES)] = excl
            return carry + jnp.max(cs)
        lax.fori_loop(0, NB // LANES, prefix_step, 0)

        def place_step(i, _):
            kvv = kc_v[pl.ds(i * LANES, LANES)]
            bv = _fold(kvv)
            rank, lastm = plsc.scan_count(bv)
            base = plsc.load_gather(cur_v, [bv])
            pos = base + rank - 1
            packed = ((kvv >> 8) << log2l) | (iota + i * LANES)
            plsc.store_scatter(kpack_s, [pos], packed)
            plsc.addupdate_scatter(cur_v, [bv], rank, mask=lastm)
            return 0
        lax.fori_loop(0, L // LANES, place_step, 0)

        BK = 4
        neg1 = jnp.full((LANES,), -1, jnp.int32)

        def prefill(i, _):
            out_v[pl.ds(i * LANES, LANES)] = neg1
            return 0
        lax.fori_loop(0, qpw * K_MAX // LANES, prefill, 0)

        def per_group(g, _):
            qv = qc_v[pl.ds(g * LANES, LANES)]
            bv = _fold(qv)
            qtop = (qv >> 8) << log2l
            combv = plsc.load_gather(comb_v, [bv])
            ptrv0 = combv >> 12
            lenv = combv & 0xFFF
            endv = ptrv0 + lenv
            out_base = (g * LANES + iota) * K_MAX
            nblocks = (jnp.max(lenv) + (BK - 1)) // BK
            cnt_s[pl.ds(0, LANES)] = ptrv0
            cnt_s[pl.ds(LANES, LANES)] = zeros

            def cond(k):
                return k < nblocks

            def block(k):
                ptrv = cnt_s[pl.ds(0, LANES)]
                cntv = cnt_s[pl.ds(LANES, LANES)]
                for _ in range(BK):
                    active = ptrv < endv
                    v = plsc.load_gather(kpack_s,
                                         [jnp.where(active, ptrv, 0)])
                    m = (active & ((v & (-L)) == qtop)
                         & (cntv < K_MAX))
                    plsc.store_scatter(out_v, [out_base + cntv],
                                       v & (L - 1), mask=m)
                    cntv = cntv + m.astype(jnp.int32)
                    ptrv = ptrv + active.astype(jnp.int32)
                cnt_s[pl.ds(0, LANES)] = ptrv
                cnt_s[pl.ds(LANES, LANES)] = cntv
                return k + 1

            lax.while_loop(cond, block, 0)
            return 0

        lax.fori_loop(0, qpw // LANES, per_group, 0)
        pltpu.sync_copy(out_v, out_hbm.at[pl.ds(gbase * K_MAX, qpw * K_MAX)])

    return pl.kernel(
        body,
        out_type=jax.ShapeDtypeStruct((n * K_MAX,), jnp.int32),
        mesh=plsc.VectorSubcoreMesh(core_axis_name="c", subcore_axis_name="s",
                                    num_cores=NUM_CORES,
                                    num_subcores=NUM_SUBCORES),
        scratch_types=[
            pltpu.VMEM((N_HASH * qpw,), jnp.float32),
            pltpu.VMEM((N_HASH * L,), jnp.float32),
            pltpu.VMEM((qpw,), jnp.int32),
            pltpu.VMEM((L,), jnp.int32),
            pltpu.VMEM((NB,), jnp.int32),
            pltpu.VMEM((NB,), jnp.int32),
            pltpu.VMEM((NB,), jnp.int32),
            pltpu.VMEM((L,), jnp.int32),
            pltpu.VMEM((2 * LANES,), jnp.int32),
            pltpu.VMEM((qpw * K_MAX,), jnp.int32),
            pltpu.SemaphoreType.DMA,
        ],
        compiler_params=pltpu.CompilerParams(needs_layout_passes=False),
    )


@jax.jit
def kernel(query, key, lsh_proj, head_idx=0):
    B, L, D = query.shape
    n = B * L
    query_bin = (query > 0).astype(jnp.float32)
    key_bin = (key > 0).astype(jnp.float32)
    qproj = query_bin.reshape(n, -1) @ lsh_proj
    kproj = key_bin.reshape(n, -1) @ lsh_proj
    out = _make_sc_call(B, L)(qproj.reshape(-1), kproj.reshape(-1))
    return out.reshape(B, L, K_MAX)


# speedup vs baseline: 33.0399x; 1.0006x over previous
import jax
import jax.numpy as jnp
from jax import lax
from jax.experimental import pallas as pl
from jax.experimental.pallas import tpu as pltpu
from jax.experimental.pallas import tpu_sc as plsc

LSH_BUCKETS = 64
LSH_BANDWIDTH = 2.0
N_HASH = 4
K_MAX = 32

NUM_CORES = 2
NUM_SUBCORES = 16
LANES = 16
NW = NUM_CORES * NUM_SUBCORES

NB = 256


def _bucket_hash(p):
    x = p / LSH_BANDWIDTH
    t = x.astype(jnp.int32)
    tf = t.astype(jnp.float32)
    f = t - jnp.where((x < 0.0) & (tf != x), 1, 0)
    return ((f % LSH_BUCKETS) + LSH_BUCKETS) % LSH_BUCKETS


def _fold(c):
    return (c ^ (c >> 8) ^ (c >> 16)) & (NB - 1)


def _make_sc_call(B, L):
    n = B * L
    qpw = n // NW
    assert qpw * NW == n and L % LANES == 0 and qpw % LANES == 0
    assert L & (L - 1) == 0
    log2l = L.bit_length() - 1

    def body(qproj_hbm, kproj_hbm, out_hbm, qp_v, qc_v, hist_v,
             comb_v, cur_v, kpack_s, cnt_s, out_v, sem):
        wid = lax.axis_index("s") * NUM_CORES + lax.axis_index("c")
        gbase = wid * qpw
        b = gbase // L
        kbase = b * L
        lbase = gbase - kbase

        copies = [
            pltpu.async_copy(qproj_hbm.at[pl.ds(gbase * N_HASH,
                                                qpw * N_HASH)],
                             qp_v.at[pl.ds(0, qpw * N_HASH)], sem),
            pltpu.async_copy(kproj_hbm.at[pl.ds(kbase * N_HASH,
                                                L * N_HASH)],
                             qp_v.at[pl.ds(qpw * N_HASH, L * N_HASH)], sem),
        ]
        for c in copies:
            c.wait()

        base_idx = lax.iota(jnp.int32, LANES) * N_HASH

        def pack_step(i, _):
            hs = [_bucket_hash(plsc.load_gather(
                      qp_v, [base_idx + (i * LANES * N_HASH + h)]))
                  for h in range(N_HASH)]
            c = ((hs[3] * LSH_BUCKETS + hs[2]) * LSH_BUCKETS
                 + hs[1]) * LSH_BUCKETS + hs[0]
            qc_v[pl.ds(i * LANES, LANES)] = c
            return 0
        lax.fori_loop(0, (qpw + L) // LANES, pack_step, 0)

        iota = lax.iota(jnp.int32, LANES)
        zeros = jnp.zeros((LANES,), jnp.int32)

        def zero_step(j, _):
            hist_v[pl.ds(j * LANES, LANES)] = zeros
            return 0
        lax.fori_loop(0, NB // LANES, zero_step, 0)

        def hist_step(i, _):
            bv = _fold(qc_v[pl.ds(qpw + i * LANES, LANES)])
            rank, lastm = plsc.scan_count(bv)
            plsc.addupdate_scatter(hist_v, [bv], rank, mask=lastm)
            return 0
        lax.fori_loop(0, L // LANES, hist_step, 0)

        def prefix_step(j, carry):
            hv = hist_v[pl.ds(j * LANES, LANES)]
            cs = plsc.cumsum(hv)
            excl = carry + cs - hv
            comb_v[pl.ds(j * LANES, LANES)] = (excl << 12) | hv
            cur_v[pl.ds(j * LANES, LANES)] = excl
            return carry + jnp.max(cs)
        lax.fori_loop(0, NB // LANES, prefix_step, 0)

        def place_step(i, _):
            kvv = qc_v[pl.ds(qpw + i * LANES, LANES)]
            bv = _fold(kvv)
            rank, lastm = plsc.scan_count(bv)
            base = plsc.load_gather(cur_v, [bv])
            pos = base + rank - 1
            packed = ((kvv >> 8) << log2l) | (iota + i * LANES)
            plsc.store_scatter(kpack_s, [pos], packed)
            plsc.addupdate_scatter(cur_v, [bv], rank, mask=lastm)
            return 0
        lax.fori_loop(0, L // LANES, place_step, 0)

        BK = 8
        neg1 = jnp.full((LANES,), -1, jnp.int32)

        def prefill(i, _):
            out_v[pl.ds(i * LANES, LANES)] = neg1
            return 0
        lax.fori_loop(0, qpw * K_MAX // LANES, prefill, 0)

        def per_group(g, _):
            qv = qc_v[pl.ds(g * LANES, LANES)]
            bv = _fold(qv)
            qtop = (qv >> 8) << log2l
            combv = plsc.load_gather(comb_v, [bv])
            ptrv0 = combv >> 12
            lenv = combv & 0xFFF
            endv = ptrv0 + lenv
            out_base = (g * LANES + iota) * K_MAX
            nblocks = (jnp.max(lenv) + (BK - 1)) // BK
            cnt_s[pl.ds(0, LANES)] = ptrv0
            cnt_s[pl.ds(LANES, LANES)] = zeros

            def cond(k):
                return k < nblocks

            def block(k):
                ptrv = cnt_s[pl.ds(0, LANES)]
                cntv = cnt_s[pl.ds(LANES, LANES)]
                for _ in range(BK):
                    active = ptrv < endv
                    v = plsc.load_gather(kpack_s,
                                         [jnp.where(active, ptrv, 0)])
                    m = (active & ((v & (-L)) == qtop)
                         & (cntv < K_MAX))
                    plsc.store_scatter(out_v, [out_base + cntv],
                                       v & (L - 1), mask=m)
                    cntv = cntv + m.astype(jnp.int32)
                    ptrv = ptrv + active.astype(jnp.int32)
                cnt_s[pl.ds(0, LANES)] = ptrv
                cnt_s[pl.ds(LANES, LANES)] = cntv
                return k + 1

            lax.while_loop(cond, block, 0)
            return 0

        lax.fori_loop(0, qpw // LANES, per_group, 0)
        pltpu.sync_copy(out_v, out_hbm.at[pl.ds(gbase * K_MAX, qpw * K_MAX)])

    return pl.kernel(
        body,
        out_type=jax.ShapeDtypeStruct((n * K_MAX,), jnp.int32),
        mesh=plsc.VectorSubcoreMesh(core_axis_name="c", subcore_axis_name="s",
                                    num_cores=NUM_CORES,
                                    num_subcores=NUM_SUBCORES),
        scratch_types=[
            pltpu.VMEM((N_HASH * (qpw + L),), jnp.float32),
            pltpu.VMEM((qpw + L,), jnp.int32),
            pltpu.VMEM((NB,), jnp.int32),
            pltpu.VMEM((NB,), jnp.int32),
            pltpu.VMEM((NB,), jnp.int32),
            pltpu.VMEM((L,), jnp.int32),
            pltpu.VMEM((2 * LANES,), jnp.int32),
            pltpu.VMEM((qpw * K_MAX,), jnp.int32),
            pltpu.SemaphoreType.DMA,
        ],
        compiler_params=pltpu.CompilerParams(needs_layout_passes=False),
    )


@jax.jit
def kernel(query, key, lsh_proj, head_idx=0):
    B, L, D = query.shape
    n = B * L
    query_bin = (query > 0).astype(jnp.float32)
    key_bin = (key > 0).astype(jnp.float32)
    qproj = query_bin.reshape(n, -1) @ lsh_proj
    kproj = key_bin.reshape(n, -1) @ lsh_proj
    out = _make_sc_call(B, L)(qproj.reshape(-1), kproj.reshape(-1))
    return out.reshape(B, L, K_MAX)


# speedup vs baseline: 33.2404x; 1.0061x over previous
import jax
import jax.numpy as jnp
from jax import lax
from jax.experimental import pallas as pl
from jax.experimental.pallas import tpu as pltpu
from jax.experimental.pallas import tpu_sc as plsc

LSH_BUCKETS = 64
LSH_BANDWIDTH = 2.0
N_HASH = 4
K_MAX = 32

NUM_CORES = 2
NUM_SUBCORES = 16
LANES = 16
NW = NUM_CORES * NUM_SUBCORES

NB = 256


def _bucket_hash(p):
    x = p / LSH_BANDWIDTH
    t = x.astype(jnp.int32)
    tf = t.astype(jnp.float32)
    f = t - jnp.where((x < 0.0) & (tf != x), 1, 0)
    return ((f % LSH_BUCKETS) + LSH_BUCKETS) % LSH_BUCKETS


def _fold(c):
    return (c ^ (c >> 8) ^ (c >> 16)) & (NB - 1)


def _make_sc_call(B, L):
    n = B * L
    qpw = n // NW
    assert qpw * NW == n and L % LANES == 0 and qpw % LANES == 0
    assert L & (L - 1) == 0
    log2l = L.bit_length() - 1

    def body(qproj_hbm, kproj_hbm, out_hbm, qp_v, qc_v, hist_v,
             comb_v, cur_v, rank_v, last_v, kpack_s, cnt_s, out_v, sem):
        wid = lax.axis_index("s") * NUM_CORES + lax.axis_index("c")
        gbase = wid * qpw
        b = gbase // L
        kbase = b * L
        lbase = gbase - kbase

        copies = [
            pltpu.async_copy(qproj_hbm.at[pl.ds(gbase * N_HASH,
                                                qpw * N_HASH)],
                             qp_v.at[pl.ds(0, qpw * N_HASH)], sem),
            pltpu.async_copy(kproj_hbm.at[pl.ds(kbase * N_HASH,
                                                L * N_HASH)],
                             qp_v.at[pl.ds(qpw * N_HASH, L * N_HASH)], sem),
        ]
        for c in copies:
            c.wait()

        base_idx = lax.iota(jnp.int32, LANES) * N_HASH

        def pack_step(i, _):
            hs = [_bucket_hash(plsc.load_gather(
                      qp_v, [base_idx + (i * LANES * N_HASH + h)]))
                  for h in range(N_HASH)]
            c = ((hs[3] * LSH_BUCKETS + hs[2]) * LSH_BUCKETS
                 + hs[1]) * LSH_BUCKETS + hs[0]
            qc_v[pl.ds(i * LANES, LANES)] = c
            return 0
        lax.fori_loop(0, (qpw + L) // LANES, pack_step, 0)

        iota = lax.iota(jnp.int32, LANES)
        zeros = jnp.zeros((LANES,), jnp.int32)

        def zero_step(j, _):
            hist_v[pl.ds(j * LANES, LANES)] = zeros
            return 0
        lax.fori_loop(0, NB // LANES, zero_step, 0)

        def hist_step(i, _):
            for j in range(2):
                off = (2 * i + j) * LANES
                bv = _fold(qc_v[pl.ds(qpw + off, LANES)])
                rank, lastm = plsc.scan_count(bv)
                plsc.addupdate_scatter(hist_v, [bv], rank, mask=lastm)
                rank_v[pl.ds(off, LANES)] = rank
                last_v[pl.ds(off, LANES)] = lastm.astype(jnp.int32)
            return 0
        lax.fori_loop(0, L // LANES // 2, hist_step, 0)

        def prefix_step(j, carry):
            hv = hist_v[pl.ds(j * LANES, LANES)]
            cs = plsc.cumsum(hv)
            excl = carry + cs - hv
            comb_v[pl.ds(j * LANES, LANES)] = (excl << 12) | hv
            cur_v[pl.ds(j * LANES, LANES)] = excl
            return carry + jnp.max(cs)
        lax.fori_loop(0, NB // LANES, prefix_step, 0)

        def place_step(i, _):
            kvv = qc_v[pl.ds(qpw + i * LANES, LANES)]
            bv = _fold(kvv)
            rank = rank_v[pl.ds(i * LANES, LANES)]
            lastm = last_v[pl.ds(i * LANES, LANES)] == 1
            base = plsc.load_gather(cur_v, [bv])
            pos = base + rank - 1
            packed = ((kvv >> 8) << log2l) | (iota + i * LANES)
            plsc.store_scatter(kpack_s, [pos], packed)
            plsc.addupdate_scatter(cur_v, [bv], rank, mask=lastm)
            return 0
        lax.fori_loop(0, L // LANES, place_step, 0)

        BK = 8
        neg1 = jnp.full((LANES,), -1, jnp.int32)

        def prefill(i, _):
            out_v[pl.ds(i * LANES, LANES)] = neg1
            return 0
        lax.fori_loop(0, qpw * K_MAX // LANES, prefill, 0)

        def per_group(g, _):
            qv = qc_v[pl.ds(g * LANES, LANES)]
            bv = _fold(qv)
            qtop = (qv >> 8) << log2l
            combv = plsc.load_gather(comb_v, [bv])
            ptrv0 = combv >> 12
            lenv = combv & 0xFFF
            endv = ptrv0 + lenv
            out_base = (g * LANES + iota) * K_MAX
            nblocks = (jnp.max(lenv) + (BK - 1)) // BK
            cnt_s[pl.ds(0, LANES)] = ptrv0
            cnt_s[pl.ds(LANES, LANES)] = zeros

            def cond(k):
                return k < nblocks

            def block(k):
                ptrv = cnt_s[pl.ds(0, LANES)]
                cntv = cnt_s[pl.ds(LANES, LANES)]
                for _ in range(BK):
                    active = ptrv < endv
                    v = plsc.load_gather(kpack_s,
                                         [jnp.where(active, ptrv, 0)])
                    m = (active & ((v & (-L)) == qtop)
                         & (cntv < K_MAX))
                    plsc.store_scatter(out_v, [out_base + cntv],
                                       v & (L - 1), mask=m)
                    cntv = cntv + m.astype(jnp.int32)
                    ptrv = ptrv + active.astype(jnp.int32)
                cnt_s[pl.ds(0, LANES)] = ptrv
                cnt_s[pl.ds(LANES, LANES)] = cntv
                return k + 1

            lax.while_loop(cond, block, 0)
            return 0

        lax.fori_loop(0, qpw // LANES, per_group, 0)
        pltpu.sync_copy(out_v, out_hbm.at[pl.ds(gbase * K_MAX, qpw * K_MAX)])

    return pl.kernel(
        body,
        out_type=jax.ShapeDtypeStruct((n * K_MAX,), jnp.int32),
        mesh=plsc.VectorSubcoreMesh(core_axis_name="c", subcore_axis_name="s",
                                    num_cores=NUM_CORES,
                                    num_subcores=NUM_SUBCORES),
        scratch_types=[
            pltpu.VMEM((N_HASH * (qpw + L),), jnp.float32),
            pltpu.VMEM((qpw + L,), jnp.int32),
            pltpu.VMEM((NB,), jnp.int32),
            pltpu.VMEM((NB,), jnp.int32),
            pltpu.VMEM((NB,), jnp.int32),
            pltpu.VMEM((L,), jnp.int32),
            pltpu.VMEM((L,), jnp.int32),
            pltpu.VMEM((L,), jnp.int32),
            pltpu.VMEM((2 * LANES,), jnp.int32),
            pltpu.VMEM((qpw * K_MAX,), jnp.int32),
            pltpu.SemaphoreType.DMA,
        ],
        compiler_params=pltpu.CompilerParams(needs_layout_passes=False),
    )


@jax.jit
def kernel(query, key, lsh_proj, head_idx=0):
    B, L, D = query.shape
    n = B * L
    query_bin = (query > 0).astype(jnp.float32)
    key_bin = (key > 0).astype(jnp.float32)
    qproj = query_bin.reshape(n, -1) @ lsh_proj
    kproj = key_bin.reshape(n, -1) @ lsh_proj
    out = _make_sc_call(B, L)(qproj.reshape(-1), kproj.reshape(-1))
    return out.reshape(B, L, K_MAX)
